# Initial kernel scaffold; baseline (speedup 1.0000x reference)
import jax
import jax.numpy as jnp
from jax.experimental import pallas as pl


def kernel(x, edge_index, W1, b1, lin_W1, lin_b1, W2, b2, lin_W2, lin_b2, Wf, bf):
    raise NotImplementedError("write your pallas kernel here")


# speedup vs baseline: 53.7777x; 53.7777x over previous
import functools

import jax
import jax.numpy as jnp
from jax import lax
from jax.experimental import pallas as pl
from jax.experimental.pallas import tpu as pltpu
from jax.experimental.pallas import tpu_sc as plsc

NC = 2
NS = 16
NW = NC * NS
CHUNK = 128
STAGE_CHUNKS = 8
STAGE_E = CHUNK * STAGE_CHUNKS
BN = 2048


def _mesh():
    return plsc.VectorSubcoreMesh(
        core_axis_name="c", subcore_axis_name="s", num_cores=NC, num_subcores=NS)


def _sc_deg(dst2, zeros1, npad, stages):
    rpt = npad // NS

    def body(dst_hbm, zeros_hbm, out0, out1, didx_v, ones_v, acc_sh):
        c = lax.axis_index("c")
        s = lax.axis_index("s")
        w = c * NS + s
        for i in range(8):
            ones_v[pl.ds(i * 16, 16)] = jnp.ones((16,), jnp.float32)
        pltpu.sync_copy(zeros_hbm.at[pl.ds(s * rpt, rpt)],
                        acc_sh.at[pl.ds(s * rpt, rpt)])
        plsc.subcore_barrier()

        base_row = w * stages * STAGE_CHUNKS

        def stage(st, carry):
            r0 = base_row + st * STAGE_CHUNKS
            pltpu.sync_copy(dst_hbm.at[pl.ds(r0, STAGE_CHUNKS)], didx_v)
            for j in range(STAGE_CHUNKS):
                pltpu.sync_copy(ones_v, acc_sh.at[didx_v.at[j]], add=True)
            return carry

        lax.fori_loop(0, stages, stage, 0)
        plsc.subcore_barrier()

        @pl.when(c == 0)
        def _():
            pltpu.sync_copy(acc_sh.at[pl.ds(s * rpt, rpt)],
                            out0.at[pl.ds(s * rpt, rpt)])

        @pl.when(c == 1)
        def _():
            pltpu.sync_copy(acc_sh.at[pl.ds(s * rpt, rpt)],
                            out1.at[pl.ds(s * rpt, rpt)])

    f = pl.kernel(
        body,
        out_type=(jax.ShapeDtypeStruct((npad,), jnp.float32),
                  jax.ShapeDtypeStruct((npad,), jnp.float32)),
        mesh=_mesh(),
        scratch_types=[
            pltpu.VMEM((STAGE_CHUNKS, CHUNK), jnp.int32),
            pltpu.VMEM((CHUNK,), jnp.float32),
            pltpu.VMEM_SHARED((npad,), jnp.float32),
        ],
        compiler_params=pltpu.CompilerParams(use_tc_tiling_on_sc=False),
    )
    return f(dst2, zeros1)


def _sc_conv(table, src2, dst2, zeros16, npad, stages):
    rpt = npad // NS

    def body(tbl_hbm, src_hbm, dst_hbm, zeros_hbm, out0, out1,
             sidx_v, didx_v, rows_v, acc_sh, sem):
        c = lax.axis_index("c")
        s = lax.axis_index("s")
        w = c * NS + s
        pltpu.sync_copy(zeros_hbm.at[pl.ds(s * rpt, rpt)],
                        acc_sh.at[pl.ds(s * rpt, rpt)])
        plsc.subcore_barrier()

        base_row = w * stages * STAGE_CHUNKS

        def stage(st, carry):
            r0 = base_row + st * STAGE_CHUNKS
            pltpu.sync_copy(src_hbm.at[pl.ds(r0, STAGE_CHUNKS)], sidx_v)
            pltpu.sync_copy(dst_hbm.at[pl.ds(r0, STAGE_CHUNKS)], didx_v)
            descs = [
                pltpu.async_copy(tbl_hbm.at[sidx_v.at[j]], rows_v.at[j], sem)
                for j in range(STAGE_CHUNKS)
            ]
            for d in descs:
                d.wait()
            for j in range(STAGE_CHUNKS):
                pltpu.sync_copy(rows_v.at[j], acc_sh.at[didx_v.at[j]], add=True)
            return carry

        lax.fori_loop(0, stages, stage, 0)
        plsc.subcore_barrier()

        @pl.when(c == 0)
        def _():
            pltpu.sync_copy(acc_sh.at[pl.ds(s * rpt, rpt)],
                            out0.at[pl.ds(s * rpt, rpt)])

        @pl.when(c == 1)
        def _():
            pltpu.sync_copy(acc_sh.at[pl.ds(s * rpt, rpt)],
                            out1.at[pl.ds(s * rpt, rpt)])

    f = pl.kernel(
        body,
        out_type=(jax.ShapeDtypeStruct((npad, 16), jnp.float32),
                  jax.ShapeDtypeStruct((npad, 16), jnp.float32)),
        mesh=_mesh(),
        scratch_types=[
            pltpu.VMEM((STAGE_CHUNKS, CHUNK), jnp.int32),
            pltpu.VMEM((STAGE_CHUNKS, CHUNK), jnp.int32),
            pltpu.VMEM((STAGE_CHUNKS, CHUNK, 16), jnp.float32),
            pltpu.VMEM_SHARED((npad, 16), jnp.float32),
            pltpu.SemaphoreType.DMA,
        ],
        compiler_params=pltpu.CompilerParams(use_tc_tiling_on_sc=False),
    )
    return f(table, src2, dst2, zeros16)


def _elu(t):
    return jnp.where(t > 0, t, jnp.exp(jnp.minimum(t, 0.0)) - 1.0)


def _tc_b(dega, degb, x, W1, npad):
    n = x.shape[0]
    grid = npad // BN

    def body(dega_r, degb_r, x_r, w1_r, dinv_r, g1_r):
        deg = dega_r[...] + degb_r[...] + 1.0
        dinv = lax.rsqrt(deg)
        hw = jnp.dot(x_r[...], w1_r[...], preferred_element_type=jnp.float32)
        dinv_r[...] = dinv
        g1_r[...] = hw * dinv[:, None]

    return pl.pallas_call(
        body,
        grid=(grid,),
        in_specs=[
            pl.BlockSpec((BN,), lambda i: (i,)),
            pl.BlockSpec((BN,), lambda i: (i,)),
            pl.BlockSpec((BN, x.shape[1]), lambda i: (i, 0)),
            pl.BlockSpec((x.shape[1], 16), lambda i: (0, 0)),
        ],
        out_specs=[
            pl.BlockSpec((BN,), lambda i: (i,)),
            pl.BlockSpec((BN, 16), lambda i: (i, 0)),
        ],
        out_shape=[
            jax.ShapeDtypeStruct((npad,), jnp.float32),
            jax.ShapeDtypeStruct((npad, 16), jnp.float32),
        ],
    )(dega, degb, x, W1)


def _tc_mid(sa, sb, dinv, g, b, lin_W, lin_b, Wnext, npad):
    grid = npad // BN

    def body(sa_r, sb_r, dinv_r, g_r, b_r, lw_r, lb_r, wn_r, h_r, gn_r):
        dinv = dinv_r[...]
        t = dinv[:, None] * (sa_r[...] + sb_r[...] + g_r[...]) + b_r[...]
        h = jnp.dot(_elu(t), lw_r[...], preferred_element_type=jnp.float32) + lb_r[...]
        gn = jnp.dot(h, wn_r[...], preferred_element_type=jnp.float32) * dinv[:, None]
        h_r[...] = h
        gn_r[...] = gn

    return pl.pallas_call(
        body,
        grid=(grid,),
        in_specs=[
            pl.BlockSpec((BN, 16), lambda i: (i, 0)),
            pl.BlockSpec((BN, 16), lambda i: (i, 0)),
            pl.BlockSpec((BN,), lambda i: (i,)),
            pl.BlockSpec((BN, 16), lambda i: (i, 0)),
            pl.BlockSpec((1, 16), lambda i: (0, 0)),
            pl.BlockSpec((16, 16), lambda i: (0, 0)),
            pl.BlockSpec((1, 16), lambda i: (0, 0)),
            pl.BlockSpec((16, 16), lambda i: (0, 0)),
        ],
        out_specs=[
            pl.BlockSpec((BN, 16), lambda i: (i, 0)),
            pl.BlockSpec((BN, 16), lambda i: (i, 0)),
        ],
        out_shape=[
            jax.ShapeDtypeStruct((npad, 16), jnp.float32),
            jax.ShapeDtypeStruct((npad, 16), jnp.float32),
        ],
    )(sa, sb, dinv, g, b.reshape(1, 16), lin_W, lin_b.reshape(1, 16), Wnext)


def _tc_fin(sa, sb, dinv, g2, b2, lin_W2, lin_b2, h1, Wf1, Wf2, bf, n, npad):
    grid = npad // BN
    C = Wf1.shape[1]

    def body(sa_r, sb_r, dinv_r, g_r, b_r, lw_r, lb_r, h1_r, wf1_r, wf2_r, bf_r, out_r):
        dinv = dinv_r[...]
        t = dinv[:, None] * (sa_r[...] + sb_r[...] + g_r[...]) + b_r[...]
        h2 = jnp.dot(_elu(t), lw_r[...], preferred_element_type=jnp.float32) + lb_r[...]
        out = (jnp.dot(h1_r[...], wf1_r[...], preferred_element_type=jnp.float32)
               + jnp.dot(h2, wf2_r[...], preferred_element_type=jnp.float32)
               + bf_r[...])
        out_r[...] = out

    return pl.pallas_call(
        body,
        grid=(grid,),
        in_specs=[
            pl.BlockSpec((BN, 16), lambda i: (i, 0)),
            pl.BlockSpec((BN, 16), lambda i: (i, 0)),
            pl.BlockSpec((BN,), lambda i: (i,)),
            pl.BlockSpec((BN, 16), lambda i: (i, 0)),
            pl.BlockSpec((1, 16), lambda i: (0, 0)),
            pl.BlockSpec((16, 16), lambda i: (0, 0)),
            pl.BlockSpec((1, 16), lambda i: (0, 0)),
            pl.BlockSpec((BN, 16), lambda i: (i, 0)),
            pl.BlockSpec((16, C), lambda i: (0, 0)),
            pl.BlockSpec((16, C), lambda i: (0, 0)),
            pl.BlockSpec((1, C), lambda i: (0, 0)),
        ],
        out_specs=pl.BlockSpec((BN, C), lambda i: (i, 0)),
        out_shape=jax.ShapeDtypeStruct((n, C), jnp.float32),
    )(sa, sb, dinv, g2, b2.reshape(1, 16), lin_W2, lin_b2.reshape(1, 16),
      h1, Wf1, Wf2, bf.reshape(1, C))


def kernel(x, edge_index, W1, b1, lin_W1, lin_b1, W2, b2, lin_W2, lin_b2, Wf, bf):
    n = x.shape[0]
    e = edge_index.shape[1]
    npad = -(-n // BN) * BN
    epw = -(-e // (NW * STAGE_E)) * STAGE_E
    ep = NW * epw
    stages = epw // STAGE_E

    src = jnp.concatenate(
        [edge_index[0], jnp.zeros((ep - e,), jnp.int32)]).reshape(ep // CHUNK, CHUNK)
    dst = jnp.concatenate(
        [edge_index[1], jnp.full((ep - e,), n, jnp.int32)]).reshape(ep // CHUNK, CHUNK)
    zeros1 = jnp.zeros((npad,), jnp.float32)
    zeros16 = jnp.zeros((npad, 16), jnp.float32)

    dega, degb = _sc_deg(dst, zeros1, npad, stages)
    dinv, g1 = _tc_b(dega, degb, x, W1, npad)
    s1a, s1b = _sc_conv(g1, src, dst, zeros16, npad, stages)
    h1, g2 = _tc_mid(s1a, s1b, dinv, g1, b1, lin_W1, lin_b1, W2, npad)
    s2a, s2b = _sc_conv(g2, src, dst, zeros16, npad, stages)
    return _tc_fin(s2a, s2b, dinv, g2, b2, lin_W2, lin_b2, h1,
                   Wf[:16], Wf[16:], bf, n, npad)


# speedup vs baseline: 70.5245x; 1.3114x over previous
import functools

import jax
import jax.numpy as jnp
from jax import lax
from jax.experimental import pallas as pl
from jax.experimental.pallas import tpu as pltpu
from jax.experimental.pallas import tpu_sc as plsc

NC = 2
NS = 16
NW = NC * NS
CHUNK = 128
SC_CONV = 4
SC_DEG = 8
STAGE_E = 2 * CHUNK * max(SC_CONV, SC_DEG)
BN = 2048
ZR = 512


def _mesh():
    return plsc.VectorSubcoreMesh(
        core_axis_name="c", subcore_axis_name="s", num_cores=NC, num_subcores=NS)


def _sc_deg(dst2, npad):
    rpt = npad // NS
    spt = (dst2.shape[0] // NW) // SC_DEG
    nz = rpt // ZR
    ztail = rpt - nz * ZR

    def body(dst_hbm, out0, out1, didx_v, ones_v, zbuf_v, acc_sh,
             semi0, semi1, sems0, sems1):
        semi = [semi0, semi1]
        sems = [sems0, sems1]
        c = lax.axis_index("c")
        s = lax.axis_index("s")
        w = c * NS + s
        base = w * spt * SC_DEG

        d0 = pltpu.async_copy(dst_hbm.at[pl.ds(base, SC_DEG)], didx_v.at[0], semi0)
        for i in range(8):
            ones_v[pl.ds(i * 16, 16)] = jnp.ones((16,), jnp.float32)

        def zfill(i, carry):
            zbuf_v[pl.ds(i * 16, 16)] = jnp.zeros((16,), jnp.float32)
            return carry
        lax.fori_loop(0, ZR // 16, zfill, 0)
        z0 = s * rpt
        for t in range(nz):
            pltpu.sync_copy(zbuf_v, acc_sh.at[pl.ds(z0 + t * ZR, ZR)])
        if ztail:
            pltpu.sync_copy(zbuf_v.at[pl.ds(0, ztail)],
                            acc_sh.at[pl.ds(z0 + nz * ZR, ztail)])
        d0.wait()
        plsc.subcore_barrier()

        def pair(i, carry):
            for p in (0, 1):
                q = 1 - p
                st = i * 2 + p
                nxt = st + 1

                @pl.when(nxt < spt)
                def _():
                    pltpu.async_copy(
                        dst_hbm.at[pl.ds(base + nxt * SC_DEG, SC_DEG)],
                        didx_v.at[q], semi[q])

                for j in range(SC_DEG):
                    pltpu.async_copy(ones_v, acc_sh.at[didx_v.at[p].at[j]],
                                     sems[p], add=True)

                @pl.when(nxt < spt)
                def _():
                    pltpu.make_async_copy(
                        dst_hbm.at[pl.ds(base + nxt * SC_DEG, SC_DEG)],
                        didx_v.at[q], semi[q]).wait()

                for j in range(SC_DEG):
                    pltpu.make_async_copy(
                        ones_v, acc_sh.at[didx_v.at[p].at[j]], sems[p]).wait()
            return carry

        lax.fori_loop(0, spt // 2, pair, 0)
        plsc.subcore_barrier()

        @pl.when(c == 0)
        def _():
            pltpu.sync_copy(acc_sh.at[pl.ds(s * rpt, rpt)],
                            out0.at[pl.ds(s * rpt, rpt)])

        @pl.when(c == 1)
        def _():
            pltpu.sync_copy(acc_sh.at[pl.ds(s * rpt, rpt)],
                            out1.at[pl.ds(s * rpt, rpt)])

    f = pl.kernel(
        body,
        out_type=(jax.ShapeDtypeStruct((npad,), jnp.float32),
                  jax.ShapeDtypeStruct((npad,), jnp.float32)),
        mesh=_mesh(),
        scratch_types=[
            pltpu.VMEM((2, SC_DEG, CHUNK), jnp.int32),
            pltpu.VMEM((CHUNK,), jnp.float32),
            pltpu.VMEM((ZR,), jnp.float32),
            pltpu.VMEM_SHARED((npad,), jnp.float32),
            pltpu.SemaphoreType.DMA,
            pltpu.SemaphoreType.DMA,
            pltpu.SemaphoreType.DMA,
            pltpu.SemaphoreType.DMA,
        ],
        compiler_params=pltpu.CompilerParams(use_tc_tiling_on_sc=False),
    )
    return f(dst2)


def _sc_conv(table, src2, dst2, npad):
    rpt = npad // NS
    spt = (src2.shape[0] // NW) // SC_CONV
    nz = rpt // ZR
    ztail = rpt - nz * ZR

    def body(tbl_hbm, src_hbm, dst_hbm, out0, out1,
             sidx_v, didx_v, rows_v, zbuf_v, acc_sh,
             semi0, semi1, semg0, semg1, sems0, sems1):
        semi = [semi0, semi1]
        semg = [semg0, semg1]
        sems = [sems0, sems1]
        c = lax.axis_index("c")
        s = lax.axis_index("s")
        w = c * NS + s
        base = w * spt * SC_CONV

        i0 = pltpu.async_copy(src_hbm.at[pl.ds(base, SC_CONV)], sidx_v.at[0], semi0)
        i1 = pltpu.async_copy(dst_hbm.at[pl.ds(base, SC_CONV)], didx_v.at[0], semi0)

        def zfill(i, carry):
            zbuf_v[i, :] = jnp.zeros((16,), jnp.float32)
            return carry
        lax.fori_loop(0, ZR, zfill, 0)
        z0 = s * rpt
        for t in range(nz):
            pltpu.sync_copy(zbuf_v, acc_sh.at[pl.ds(z0 + t * ZR, ZR)])
        if ztail:
            pltpu.sync_copy(zbuf_v.at[pl.ds(0, ztail)],
                            acc_sh.at[pl.ds(z0 + nz * ZR, ztail)])

        i0.wait()
        i1.wait()
        for j in range(SC_CONV):
            pltpu.async_copy(tbl_hbm.at[sidx_v.at[0].at[j]], rows_v.at[0, j],
                             semg0)
        plsc.subcore_barrier()

        def pair(i, carry):
            for p in (0, 1):
                q = 1 - p
                st = i * 2 + p
                nxt = st + 1
                rnx = base + nxt * SC_CONV

                @pl.when(nxt < spt)
                def _():
                    pltpu.async_copy(src_hbm.at[pl.ds(rnx, SC_CONV)],
                                     sidx_v.at[q], semi[q])
                    pltpu.async_copy(dst_hbm.at[pl.ds(rnx, SC_CONV)],
                                     didx_v.at[q], semi[q])

                for j in range(SC_CONV):
                    pltpu.make_async_copy(tbl_hbm.at[sidx_v.at[p].at[j]],
                                          rows_v.at[p, j], semg[p]).wait()
                for j in range(SC_CONV):
                    pltpu.async_copy(rows_v.at[p, j],
                                     acc_sh.at[didx_v.at[p].at[j]],
                                     sems[p], add=True)

                @pl.when(nxt < spt)
                def _():
                    pltpu.make_async_copy(src_hbm.at[pl.ds(rnx, SC_CONV)],
                                          sidx_v.at[q], semi[q]).wait()
                    pltpu.make_async_copy(dst_hbm.at[pl.ds(rnx, SC_CONV)],
                                          didx_v.at[q], semi[q]).wait()
                    for j in range(SC_CONV):
                        pltpu.async_copy(tbl_hbm.at[sidx_v.at[q].at[j]],
                                         rows_v.at[q, j], semg[q])

                for j in range(SC_CONV):
                    pltpu.make_async_copy(rows_v.at[p, j],
                                          acc_sh.at[didx_v.at[p].at[j]],
                                          sems[p]).wait()
            return carry

        lax.fori_loop(0, spt // 2, pair, 0)
        plsc.subcore_barrier()

        @pl.when(c == 0)
        def _():
            pltpu.sync_copy(acc_sh.at[pl.ds(s * rpt, rpt)],
                            out0.at[pl.ds(s * rpt, rpt)])

        @pl.when(c == 1)
        def _():
            pltpu.sync_copy(acc_sh.at[pl.ds(s * rpt, rpt)],
                            out1.at[pl.ds(s * rpt, rpt)])

    f = pl.kernel(
        body,
        out_type=(jax.ShapeDtypeStruct((npad, 16), jnp.float32),
                  jax.ShapeDtypeStruct((npad, 16), jnp.float32)),
        mesh=_mesh(),
        scratch_types=[
            pltpu.VMEM((2, SC_CONV, CHUNK), jnp.int32),
            pltpu.VMEM((2, SC_CONV, CHUNK), jnp.int32),
            pltpu.VMEM((2, SC_CONV, CHUNK, 16), jnp.float32),
            pltpu.VMEM((ZR, 16), jnp.float32),
            pltpu.VMEM_SHARED((npad, 16), jnp.float32),
            pltpu.SemaphoreType.DMA,
            pltpu.SemaphoreType.DMA,
            pltpu.SemaphoreType.DMA,
            pltpu.SemaphoreType.DMA,
            pltpu.SemaphoreType.DMA,
            pltpu.SemaphoreType.DMA,
        ],
        compiler_params=pltpu.CompilerParams(use_tc_tiling_on_sc=False),
    )
    return f(table, src2, dst2)


def _elu(t):
    return jnp.where(t > 0, t, jnp.exp(jnp.minimum(t, 0.0)) - 1.0)


def _tc_b(dega, degb, x, W1, npad):
    n = x.shape[0]
    grid = npad // BN

    def body(dega_r, degb_r, x_r, w1_r, dinv_r, g1_r):
        deg = dega_r[...] + degb_r[...] + 1.0
        dinv = lax.rsqrt(deg)
        hw = jnp.dot(x_r[...], w1_r[...], preferred_element_type=jnp.float32)
        dinv_r[...] = dinv
        g1_r[...] = hw * dinv[:, None]

    return pl.pallas_call(
        body,
        grid=(grid,),
        in_specs=[
            pl.BlockSpec((BN,), lambda i: (i,)),
            pl.BlockSpec((BN,), lambda i: (i,)),
            pl.BlockSpec((BN, x.shape[1]), lambda i: (i, 0)),
            pl.BlockSpec((x.shape[1], 16), lambda i: (0, 0)),
        ],
        out_specs=[
            pl.BlockSpec((BN,), lambda i: (i,)),
            pl.BlockSpec((BN, 16), lambda i: (i, 0)),
        ],
        out_shape=[
            jax.ShapeDtypeStruct((npad,), jnp.float32),
            jax.ShapeDtypeStruct((npad, 16), jnp.float32),
        ],
    )(dega, degb, x, W1)


def _tc_mid(sa, sb, dinv, g, b, lin_W, lin_b, Wnext, npad):
    grid = npad // BN

    def body(sa_r, sb_r, dinv_r, g_r, b_r, lw_r, lb_r, wn_r, h_r, gn_r):
        dinv = dinv_r[...]
        t = dinv[:, None] * (sa_r[...] + sb_r[...] + g_r[...]) + b_r[...]
        h = jnp.dot(_elu(t), lw_r[...], preferred_element_type=jnp.float32) + lb_r[...]
        gn = jnp.dot(h, wn_r[...], preferred_element_type=jnp.float32) * dinv[:, None]
        h_r[...] = h
        gn_r[...] = gn

    return pl.pallas_call(
        body,
        grid=(grid,),
        in_specs=[
            pl.BlockSpec((BN, 16), lambda i: (i, 0)),
            pl.BlockSpec((BN, 16), lambda i: (i, 0)),
            pl.BlockSpec((BN,), lambda i: (i,)),
            pl.BlockSpec((BN, 16), lambda i: (i, 0)),
            pl.BlockSpec((1, 16), lambda i: (0, 0)),
            pl.BlockSpec((16, 16), lambda i: (0, 0)),
            pl.BlockSpec((1, 16), lambda i: (0, 0)),
            pl.BlockSpec((16, 16), lambda i: (0, 0)),
        ],
        out_specs=[
            pl.BlockSpec((BN, 16), lambda i: (i, 0)),
            pl.BlockSpec((BN, 16), lambda i: (i, 0)),
        ],
        out_shape=[
            jax.ShapeDtypeStruct((npad, 16), jnp.float32),
            jax.ShapeDtypeStruct((npad, 16), jnp.float32),
        ],
    )(sa, sb, dinv, g, b.reshape(1, 16), lin_W, lin_b.reshape(1, 16), Wnext)


def _tc_fin(sa, sb, dinv, g2, b2, lin_W2, lin_b2, h1, Wf1, Wf2, bf, n, npad):
    grid = npad // BN
    C = Wf1.shape[1]

    def body(sa_r, sb_r, dinv_r, g_r, b_r, lw_r, lb_r, h1_r, wf1_r, wf2_r, bf_r, out_r):
        dinv = dinv_r[...]
        t = dinv[:, None] * (sa_r[...] + sb_r[...] + g_r[...]) + b_r[...]
        h2 = jnp.dot(_elu(t), lw_r[...], preferred_element_type=jnp.float32) + lb_r[...]
        out = (jnp.dot(h1_r[...], wf1_r[...], preferred_element_type=jnp.float32)
               + jnp.dot(h2, wf2_r[...], preferred_element_type=jnp.float32)
               + bf_r[...])
        out_r[...] = out

    return pl.pallas_call(
        body,
        grid=(grid,),
        in_specs=[
            pl.BlockSpec((BN, 16), lambda i: (i, 0)),
            pl.BlockSpec((BN, 16), lambda i: (i, 0)),
            pl.BlockSpec((BN,), lambda i: (i,)),
            pl.BlockSpec((BN, 16), lambda i: (i, 0)),
            pl.BlockSpec((1, 16), lambda i: (0, 0)),
            pl.BlockSpec((16, 16), lambda i: (0, 0)),
            pl.BlockSpec((1, 16), lambda i: (0, 0)),
            pl.BlockSpec((BN, 16), lambda i: (i, 0)),
            pl.BlockSpec((16, C), lambda i: (0, 0)),
            pl.BlockSpec((16, C), lambda i: (0, 0)),
            pl.BlockSpec((1, C), lambda i: (0, 0)),
        ],
        out_specs=pl.BlockSpec((BN, C), lambda i: (i, 0)),
        out_shape=jax.ShapeDtypeStruct((n, C), jnp.float32),
    )(sa, sb, dinv, g2, b2.reshape(1, 16), lin_W2, lin_b2.reshape(1, 16),
      h1, Wf1, Wf2, bf.reshape(1, C))


def kernel(x, edge_index, W1, b1, lin_W1, lin_b1, W2, b2, lin_W2, lin_b2, Wf, bf):
    n = x.shape[0]
    e = edge_index.shape[1]
    npad = -(-n // BN) * BN
    epw = -(-e // (NW * STAGE_E)) * STAGE_E
    ep = NW * epw

    src = jnp.concatenate(
        [edge_index[0], jnp.zeros((ep - e,), jnp.int32)]).reshape(ep // CHUNK, CHUNK)
    dst = jnp.concatenate(
        [edge_index[1], jnp.full((ep - e,), n, jnp.int32)]).reshape(ep // CHUNK, CHUNK)

    dega, degb = _sc_deg(dst, npad)
    dinv, g1 = _tc_b(dega, degb, x, W1, npad)
    s1a, s1b = _sc_conv(g1, src, dst, npad)
    h1, g2 = _tc_mid(s1a, s1b, dinv, g1, b1, lin_W1, lin_b1, W2, npad)
    s2a, s2b = _sc_conv(g2, src, dst, npad)
    return _tc_fin(s2a, s2b, dinv, g2, b2, lin_W2, lin_b2, h1,
                   Wf[:16], Wf[16:], bf, n, npad)


# speedup vs baseline: 82.4951x; 1.1697x over previous
import functools

import jax
import jax.numpy as jnp
from jax import lax
from jax.experimental import pallas as pl
from jax.experimental.pallas import tpu as pltpu
from jax.experimental.pallas import tpu_sc as plsc

NC = 2
NS = 16
NW = NC * NS
CHUNK = 128
SC_CONV = 4
SC_DEG = 8
STAGE_E = 2 * CHUNK * max(SC_CONV, SC_DEG)
BN = 2048
ZR = 512


def _mesh():
    return plsc.VectorSubcoreMesh(
        core_axis_name="c", subcore_axis_name="s", num_cores=NC, num_subcores=NS)


def _sc_deg(dst2, npad):
    rpt = npad // NS
    spt = (dst2.shape[0] // NW) // SC_DEG
    nz = rpt // ZR
    ztail = rpt - nz * ZR

    def body(dst_hbm, out0, out1, didx_v, ones_v, zbuf_v, acc_sh,
             semi0, semi1, sems0, sems1):
        semi = [semi0, semi1]
        sems = [sems0, sems1]
        c = lax.axis_index("c")
        s = lax.axis_index("s")
        w = c * NS + s
        base = w * spt * SC_DEG

        d0 = pltpu.async_copy(dst_hbm.at[pl.ds(base, SC_DEG)], didx_v.at[0], semi0)
        for i in range(8):
            ones_v[pl.ds(i * 16, 16)] = jnp.ones((16,), jnp.float32)

        def zfill(i, carry):
            zbuf_v[pl.ds(i * 16, 16)] = jnp.zeros((16,), jnp.float32)
            return carry
        lax.fori_loop(0, ZR // 16, zfill, 0)
        z0 = s * rpt
        for t in range(nz):
            pltpu.sync_copy(zbuf_v, acc_sh.at[pl.ds(z0 + t * ZR, ZR)])
        if ztail:
            pltpu.sync_copy(zbuf_v.at[pl.ds(0, ztail)],
                            acc_sh.at[pl.ds(z0 + nz * ZR, ztail)])
        d0.wait()
        plsc.subcore_barrier()

        def pair(i, carry):
            for p in (0, 1):
                q = 1 - p
                st = i * 2 + p
                nxt = st + 1

                @pl.when(nxt < spt)
                def _():
                    pltpu.async_copy(
                        dst_hbm.at[pl.ds(base + nxt * SC_DEG, SC_DEG)],
                        didx_v.at[q], semi[q])

                for j in range(SC_DEG):
                    pltpu.async_copy(ones_v, acc_sh.at[didx_v.at[p].at[j]],
                                     sems[p], add=True)

                @pl.when(nxt < spt)
                def _():
                    pltpu.make_async_copy(
                        dst_hbm.at[pl.ds(base + nxt * SC_DEG, SC_DEG)],
                        didx_v.at[q], semi[q]).wait()

                for j in range(SC_DEG):
                    pltpu.make_async_copy(
                        ones_v, acc_sh.at[didx_v.at[p].at[j]], sems[p]).wait()
            return carry

        lax.fori_loop(0, spt // 2, pair, 0)
        plsc.subcore_barrier()

        @pl.when(c == 0)
        def _():
            pltpu.sync_copy(acc_sh.at[pl.ds(s * rpt, rpt)],
                            out0.at[pl.ds(s * rpt, rpt)])

        @pl.when(c == 1)
        def _():
            pltpu.sync_copy(acc_sh.at[pl.ds(s * rpt, rpt)],
                            out1.at[pl.ds(s * rpt, rpt)])

    f = pl.kernel(
        body,
        out_type=(jax.ShapeDtypeStruct((npad,), jnp.float32),
                  jax.ShapeDtypeStruct((npad,), jnp.float32)),
        mesh=_mesh(),
        scratch_types=[
            pltpu.VMEM((2, SC_DEG, CHUNK), jnp.int32),
            pltpu.VMEM((CHUNK,), jnp.float32),
            pltpu.VMEM((ZR,), jnp.float32),
            pltpu.VMEM_SHARED((npad,), jnp.float32),
            pltpu.SemaphoreType.DMA,
            pltpu.SemaphoreType.DMA,
            pltpu.SemaphoreType.DMA,
            pltpu.SemaphoreType.DMA,
        ],
        compiler_params=pltpu.CompilerParams(use_tc_tiling_on_sc=False),
    )
    return f(dst2)


def _sc_conv(table, src2, dst2, npad):
    rpt = npad // NS
    spt = (src2.shape[0] // NW) // SC_CONV
    nz = rpt // ZR
    ztail = rpt - nz * ZR

    def body(tbl_hbm, src_hbm, dst_hbm, out0, out1,
             sidx_v, didx_v, rows_v, zbuf_v, acc_sh,
             semi0, semi1, semg0, semg1, sems0, sems1):
        semi = [semi0, semi1]
        semg = [semg0, semg1]
        sems = [sems0, sems1]
        c = lax.axis_index("c")
        s = lax.axis_index("s")
        w = c * NS + s
        base = w * spt * SC_CONV

        i0 = pltpu.async_copy(src_hbm.at[pl.ds(base, SC_CONV)], sidx_v.at[0], semi0)
        i1 = pltpu.async_copy(dst_hbm.at[pl.ds(base, SC_CONV)], didx_v.at[0], semi0)

        def zfill(i, carry):
            zbuf_v[i, :] = jnp.zeros((16,), jnp.float32)
            return carry
        lax.fori_loop(0, ZR, zfill, 0)
        z0 = s * rpt
        for t in range(nz):
            pltpu.sync_copy(zbuf_v, acc_sh.at[pl.ds(z0 + t * ZR, ZR)])
        if ztail:
            pltpu.sync_copy(zbuf_v.at[pl.ds(0, ztail)],
                            acc_sh.at[pl.ds(z0 + nz * ZR, ztail)])

        i0.wait()
        i1.wait()
        for j in range(SC_CONV):
            pltpu.async_copy(tbl_hbm.at[sidx_v.at[0].at[j]], rows_v.at[0, j],
                             semg0)
        plsc.subcore_barrier()

        def pair(i, carry):
            for p in (0, 1):
                q = 1 - p
                st = i * 2 + p
                nxt = st + 1
                rnx = base + nxt * SC_CONV

                @pl.when(nxt < spt)
                def _():
                    pltpu.async_copy(src_hbm.at[pl.ds(rnx, SC_CONV)],
                                     sidx_v.at[q], semi[q])
                    pltpu.async_copy(dst_hbm.at[pl.ds(rnx, SC_CONV)],
                                     didx_v.at[q], semi[q])

                for j in range(SC_CONV):
                    pltpu.make_async_copy(tbl_hbm.at[sidx_v.at[p].at[j]],
                                          rows_v.at[p, j], semg[p]).wait()
                for j in range(SC_CONV):
                    pltpu.async_copy(rows_v.at[p, j],
                                     acc_sh.at[didx_v.at[p].at[j]],
                                     sems[p], add=True)

                @pl.when(nxt < spt)
                def _():
                    pltpu.make_async_copy(src_hbm.at[pl.ds(rnx, SC_CONV)],
                                          sidx_v.at[q], semi[q]).wait()
                    pltpu.make_async_copy(dst_hbm.at[pl.ds(rnx, SC_CONV)],
                                          didx_v.at[q], semi[q]).wait()
                    for j in range(SC_CONV):
                        pltpu.async_copy(tbl_hbm.at[sidx_v.at[q].at[j]],
                                         rows_v.at[q, j], semg[q])

                for j in range(SC_CONV):
                    pltpu.make_async_copy(rows_v.at[p, j],
                                          acc_sh.at[didx_v.at[p].at[j]],
                                          sems[p]).wait()
            return carry

        lax.fori_loop(0, spt // 2, pair, 0)
        plsc.subcore_barrier()

        @pl.when(c == 0)
        def _():
            pltpu.sync_copy(acc_sh.at[pl.ds(s * rpt, rpt)],
                            out0.at[pl.ds(s * rpt, rpt)])

        @pl.when(c == 1)
        def _():
            pltpu.sync_copy(acc_sh.at[pl.ds(s * rpt, rpt)],
                            out1.at[pl.ds(s * rpt, rpt)])

    f = pl.kernel(
        body,
        out_type=(jax.ShapeDtypeStruct((npad, 16), jnp.float32),
                  jax.ShapeDtypeStruct((npad, 16), jnp.float32)),
        mesh=_mesh(),
        scratch_types=[
            pltpu.VMEM((2, SC_CONV, CHUNK), jnp.int32),
            pltpu.VMEM((2, SC_CONV, CHUNK), jnp.int32),
            pltpu.VMEM((2, SC_CONV, CHUNK, 16), jnp.float32),
            pltpu.VMEM((ZR, 16), jnp.float32),
            pltpu.VMEM_SHARED((npad, 16), jnp.float32),
            pltpu.SemaphoreType.DMA,
            pltpu.SemaphoreType.DMA,
            pltpu.SemaphoreType.DMA,
            pltpu.SemaphoreType.DMA,
            pltpu.SemaphoreType.DMA,
            pltpu.SemaphoreType.DMA,
        ],
        compiler_params=pltpu.CompilerParams(use_tc_tiling_on_sc=False),
    )
    return f(table, src2, dst2)


def _elu(t):
    return jnp.where(t > 0, t, jnp.exp(jnp.minimum(t, 0.0)) - 1.0)


BP = BN // 8


def _tc_xw(xp, W1p, npad8):
    kdim = xp.shape[1]

    def body(xp_r, w_r, o_r):
        o_r[...] = jnp.dot(xp_r[...], w_r[...], preferred_element_type=jnp.float32)

    return pl.pallas_call(
        body,
        grid=(npad8 // BP,),
        in_specs=[
            pl.BlockSpec((BP, kdim), lambda i: (i, 0)),
            pl.BlockSpec((kdim, 128), lambda i: (0, 0)),
        ],
        out_specs=pl.BlockSpec((BP, 128), lambda i: (i, 0)),
        out_shape=jax.ShapeDtypeStruct((npad8, 128), jnp.float32),
    )(xp, W1p)


def _tc_b(dega8, degb8, hw1p, S, npad8):
    def body(da_r, db_r, hw_r, s_r, dinvp_r, g1p_r):
        dinv8 = lax.rsqrt(da_r[...] + db_r[...] + 1.0)
        dinvp = jnp.dot(dinv8, s_r[...], preferred_element_type=jnp.float32)
        dinvp_r[...] = dinvp
        g1p_r[...] = hw_r[...] * dinvp

    return pl.pallas_call(
        body,
        grid=(npad8 // BP,),
        in_specs=[
            pl.BlockSpec((BP, 8), lambda i: (i, 0)),
            pl.BlockSpec((BP, 8), lambda i: (i, 0)),
            pl.BlockSpec((BP, 128), lambda i: (i, 0)),
            pl.BlockSpec((8, 128), lambda i: (0, 0)),
        ],
        out_specs=[
            pl.BlockSpec((BP, 128), lambda i: (i, 0)),
            pl.BlockSpec((BP, 128), lambda i: (i, 0)),
        ],
        out_shape=[
            jax.ShapeDtypeStruct((npad8, 128), jnp.float32),
            jax.ShapeDtypeStruct((npad8, 128), jnp.float32),
        ],
    )(dega8, degb8, hw1p, S)


def _tc_mid(sap, sbp, dinvp, gp, bp, lin_Wp, lin_bp, Wnp, npad8):
    def body(sa_r, sb_r, dv_r, g_r, b_r, lw_r, lb_r, wn_r, h_r, gn_r):
        dinvp = dv_r[...]
        t = dinvp * (sa_r[...] + sb_r[...] + g_r[...]) + b_r[...]
        h = jnp.dot(_elu(t), lw_r[...], preferred_element_type=jnp.float32) + lb_r[...]
        gn = jnp.dot(h, wn_r[...], preferred_element_type=jnp.float32) * dinvp
        h_r[...] = h
        gn_r[...] = gn

    return pl.pallas_call(
        body,
        grid=(npad8 // BP,),
        in_specs=[
            pl.BlockSpec((BP, 128), lambda i: (i, 0)),
            pl.BlockSpec((BP, 128), lambda i: (i, 0)),
            pl.BlockSpec((BP, 128), lambda i: (i, 0)),
            pl.BlockSpec((BP, 128), lambda i: (i, 0)),
            pl.BlockSpec((1, 128), lambda i: (0, 0)),
            pl.BlockSpec((128, 128), lambda i: (0, 0)),
            pl.BlockSpec((1, 128), lambda i: (0, 0)),
            pl.BlockSpec((128, 128), lambda i: (0, 0)),
        ],
        out_specs=[
            pl.BlockSpec((BP, 128), lambda i: (i, 0)),
            pl.BlockSpec((BP, 128), lambda i: (i, 0)),
        ],
        out_shape=[
            jax.ShapeDtypeStruct((npad8, 128), jnp.float32),
            jax.ShapeDtypeStruct((npad8, 128), jnp.float32),
        ],
    )(sap, sbp, dinvp, gp, bp, lin_Wp, lin_bp, Wnp)


def _tc_fin(sap, sbp, dinvp, g2p, b2p, lin_W2p, lin_b2p, h1p, Wf1p, Wf2p, bfp,
            npad8):
    cp = Wf1p.shape[1]

    def body(sa_r, sb_r, dv_r, g_r, b_r, lw_r, lb_r, h1_r, w1_r, w2_r, bf_r, o_r):
        dinvp = dv_r[...]
        t = dinvp * (sa_r[...] + sb_r[...] + g_r[...]) + b_r[...]
        h2 = jnp.dot(_elu(t), lw_r[...], preferred_element_type=jnp.float32) + lb_r[...]
        o_r[...] = (jnp.dot(h1_r[...], w1_r[...], preferred_element_type=jnp.float32)
                    + jnp.dot(h2, w2_r[...], preferred_element_type=jnp.float32)
                    + bf_r[...])

    return pl.pallas_call(
        body,
        grid=(npad8 // BP,),
        in_specs=[
            pl.BlockSpec((BP, 128), lambda i: (i, 0)),
            pl.BlockSpec((BP, 128), lambda i: (i, 0)),
            pl.BlockSpec((BP, 128), lambda i: (i, 0)),
            pl.BlockSpec((BP, 128), lambda i: (i, 0)),
            pl.BlockSpec((1, 128), lambda i: (0, 0)),
            pl.BlockSpec((128, 128), lambda i: (0, 0)),
            pl.BlockSpec((1, 128), lambda i: (0, 0)),
            pl.BlockSpec((BP, 128), lambda i: (i, 0)),
            pl.BlockSpec((128, cp), lambda i: (0, 0)),
            pl.BlockSpec((128, cp), lambda i: (0, 0)),
            pl.BlockSpec((1, cp), lambda i: (0, 0)),
        ],
        out_specs=pl.BlockSpec((BP, cp), lambda i: (i, 0)),
        out_shape=jax.ShapeDtypeStruct((npad8, cp), jnp.float32),
    )(sap, sbp, dinvp, g2p, b2p, lin_W2p, lin_b2p, h1p, Wf1p, Wf2p, bfp)


def kernel(x, edge_index, W1, b1, lin_W1, lin_b1, W2, b2, lin_W2, lin_b2, Wf, bf):
    n = x.shape[0]
    d = x.shape[1]
    e = edge_index.shape[1]
    C = Wf.shape[1]
    npad = -(-n // BN) * BN
    npad8 = npad // 8
    epw = -(-e // (NW * STAGE_E)) * STAGE_E
    ep = NW * epw

    src = jnp.concatenate(
        [edge_index[0], jnp.zeros((ep - e,), jnp.int32)]).reshape(ep // CHUNK, CHUNK)
    dst = jnp.concatenate(
        [edge_index[1], jnp.full((ep - e,), n, jnp.int32)]).reshape(ep // CHUNK, CHUNK)

    i8 = jnp.eye(8, dtype=jnp.float32)
    xp = x.reshape(n // 8, 8 * d)
    W1p = jnp.kron(i8, W1)
    lin_W1p = jnp.kron(i8, lin_W1)
    W2p = jnp.kron(i8, W2)
    lin_W2p = jnp.kron(i8, lin_W2)
    Wf1p = jnp.kron(i8, Wf[:16])
    Wf2p = jnp.kron(i8, Wf[16:])
    b1p = jnp.tile(b1, 8).reshape(1, 128)
    lin_b1p = jnp.tile(lin_b1, 8).reshape(1, 128)
    b2p = jnp.tile(b2, 8).reshape(1, 128)
    lin_b2p = jnp.tile(lin_b2, 8).reshape(1, 128)
    bfp = jnp.tile(bf, 8).reshape(1, 8 * C)
    S = jnp.repeat(i8, 16, axis=1)

    hw1p = _tc_xw(xp, W1p, npad8)
    dega, degb = _sc_deg(dst, npad)
    dinvp, g1p = _tc_b(dega.reshape(npad8, 8), degb.reshape(npad8, 8),
                       hw1p, S, npad8)
    s1a, s1b = _sc_conv(g1p.reshape(npad, 16), src, dst, npad)
    h1p, g2p = _tc_mid(s1a.reshape(npad8, 128), s1b.reshape(npad8, 128),
                       dinvp, g1p, b1p, lin_W1p, lin_b1p, W2p, npad8)
    s2a, s2b = _sc_conv(g2p.reshape(npad, 16), src, dst, npad)
    outp = _tc_fin(s2a.reshape(npad8, 128), s2b.reshape(npad8, 128),
                   dinvp, g2p, b2p, lin_W2p, lin_b2p, h1p, Wf1p, Wf2p, bfp,
                   npad8)
    return outp.reshape(npad, C)[:n]
